# Initial kernel scaffold; baseline (speedup 1.0000x reference)
import jax
import jax.numpy as jnp
from jax.experimental import pallas as pl


def kernel(p3, p4, p5, p6, p7, cls_w, cls_b, cls_gn_g, cls_gn_b, bbox_w, bbox_b, bbox_gn_g, bbox_gn_b, head_cls_w, head_cls_b, head_bbox_w, head_bbox_b, head_ctr_w, head_ctr_b, scales):
    raise NotImplementedError("write your pallas kernel here")


# speedup vs baseline: 1.3615x; 1.3615x over previous
import functools
import math

import jax
import jax.numpy as jnp
import numpy as np
from jax.experimental import pallas as pl
from jax.experimental.pallas import tpu as pltpu

C = 256
NCONV = 4
GROUPS = 32
EPS = 1e-5
LEVEL_HW = (64, 32, 16, 8, 4)

ACT_DT = jnp.float32
MM_DT = jnp.float32

GEOM = []
for _hw in LEVEL_HW:
    _Hp, _Wp = _hw + 2, _hw + 2
    _Np = _Hp * _Wp
    _S = _Wp + 1
    GEOM.append((_hw, _hw, _Hp, _Wp, _Np, _S, _Np + 2 * _S))
TMAX = max(g[6] for g in GEOM)


def _fcos_kernel(*refs):
    i = 0
    x_refs = refs[i:i + 5]; i += 5
    (cls_tw, cls_bb, cls_gg, cls_be,
     box_tw, box_bb, box_gg, box_be,
     hcls_w, hcls_b, hbc_w, hbc_b, gmat, svec, rsel) = refs[i:i + 15]
    i += 15
    m_refs = refs[i:i + 5]; i += 5
    lo_refs = refs[i:i + 5]; i += 5
    bo_refs = refs[i:i + 5]; i += 5
    X = refs[i]

    gmat_v = gmat[...]
    rsel_v = rsel[...]

    for l, (H, W, Hp, Wp, Np, S, T) in enumerate(GEOM):
        offs = [dy * Wp + dx for dy in (-1, 0, 1) for dx in (-1, 0, 1)]
        inv_cnt = 1.0 / float(H * W * (C // GROUPS))
        mask = m_refs[l][...]

        def conv9(wref, base, offs=offs, Np=Np, S=S):
            acc = None
            for t, off in enumerate(offs):
                xs = X[pl.ds(S + off, Np), :]
                p = jnp.dot(xs, wref[base + t],
                            preferred_element_type=jnp.float32)
                acc = p if acc is None else acc + p
            return acc

        def run_tower(twref, bref, gref, beref,
                      l=l, Np=Np, S=S, T=T, mask=mask, inv_cnt=inv_cnt,
                      conv9=conv9):
            X[pl.ds(0, T), :] = jnp.zeros((T, C), ACT_DT)
            X[pl.ds(S, Np), :] = x_refs[l][...]

            def body(i, carry):
                acc = conv9(twref, i * 9)
                y = (acc + bref[i]) * mask
                s1 = jnp.sum(y, axis=0, keepdims=True)
                s2 = jnp.sum(y * y, axis=0, keepdims=True)
                gs1 = jnp.dot(s1, gmat_v, preferred_element_type=jnp.float32)
                gs2 = jnp.dot(s2, gmat_v, preferred_element_type=jnp.float32)
                mean = gs1 * inv_cnt
                var = gs2 * inv_cnt - mean * mean
                rstd = jax.lax.rsqrt(var + EPS)
                z = (y - mean) * (rstd * gref[i]) + beref[i]
                z = jnp.maximum(z, 0.0) * mask
                X[pl.ds(S, Np), :] = z.astype(ACT_DT)
                return carry

            jax.lax.fori_loop(0, NCONV, body, 0, unroll=True)

        run_tower(cls_tw, cls_bb, cls_gg, cls_be)
        lo_refs[l][...] = conv9(hcls_w, 0) + hcls_b[...]

        run_tower(box_tw, box_bb, box_gg, box_be)
        v = (conv9(hbc_w, 0) + hbc_b[...]) * svec[l]
        bo_refs[l][...] = rsel_v * jnp.maximum(v, 0.0) + (1.0 - rsel_v) * v


def _tower_taps(w):
    return w.transpose(0, 3, 4, 2, 1).reshape(NCONV * 9, C, C).astype(MM_DT)


def _head_taps(w, pad_to=128):
    o = w.shape[0]
    t = w.transpose(2, 3, 1, 0).reshape(9, C, o)
    return jnp.pad(t, ((0, 0), (0, 0), (0, pad_to - o))).astype(MM_DT)


@functools.partial(jax.jit)
def kernel(p3, p4, p5, p6, p7,
           cls_w, cls_b, cls_gn_g, cls_gn_b,
           bbox_w, bbox_b, bbox_gn_g, bbox_gn_b,
           head_cls_w, head_cls_b, head_bbox_w, head_bbox_b,
           head_ctr_w, head_ctr_b, scales):
    feats = (p3, p4, p5, p6, p7)
    xs, masks = [], []
    for (H, W, Hp, Wp, Np, S, T), f in zip(GEOM, feats):
        x = jnp.pad(f[0].transpose(1, 2, 0), ((1, 1), (1, 1), (0, 0)))
        xs.append(x.reshape(Np, C).astype(ACT_DT))
        m = np.zeros((Hp, Wp, 1), np.float32)
        m[1:H + 1, 1:W + 1] = 1.0
        masks.append(jnp.asarray(m.reshape(Np, 1)))

    cls_tw = _tower_taps(cls_w)
    box_tw = _tower_taps(bbox_w)
    per_layer = lambda a: a.reshape(NCONV, 1, C).astype(jnp.float32)
    cls_bb, cls_gg, cls_be = per_layer(cls_b), per_layer(cls_gn_g), per_layer(cls_gn_b)
    box_bb, box_gg, box_be = per_layer(bbox_b), per_layer(bbox_gn_g), per_layer(bbox_gn_b)

    hcls_w = _head_taps(head_cls_w)
    hcls_b = jnp.pad(head_cls_b, (0, 128 - 80)).reshape(1, 128).astype(jnp.float32)
    hbc_w = _head_taps(jnp.concatenate([head_bbox_w, head_ctr_w], axis=0))
    hbc_b = jnp.pad(jnp.concatenate([head_bbox_b, head_ctr_b]), (0, 128 - 5)
                    ).reshape(1, 128).astype(jnp.float32)

    gmat = jnp.asarray(np.kron(np.eye(GROUPS, dtype=np.float32),
                               np.ones((C // GROUPS, C // GROUPS), np.float32)))
    lane = np.arange(128)
    svec = jnp.where(jnp.asarray(lane[None, None, :] < 4),
                     scales[:, None, None].astype(jnp.float32), 1.0)
    rsel = jnp.asarray((lane[None, :] < 4).astype(np.float32))

    out_shape = ([jax.ShapeDtypeStruct((g[4], 128), jnp.float32) for g in GEOM]
                 + [jax.ShapeDtypeStruct((g[4], 128), jnp.float32) for g in GEOM])

    outs = pl.pallas_call(
        _fcos_kernel,
        out_shape=out_shape,
        scratch_shapes=[pltpu.VMEM((TMAX, C), ACT_DT)],
    )(*xs, cls_tw, cls_bb, cls_gg, cls_be, box_tw, box_bb, box_gg, box_be,
      hcls_w, hcls_b, hbc_w, hbc_b, gmat, svec, rsel, *masks)

    logits, bboxs, ctrs = [], [], []
    for l, (H, W, Hp, Wp, Np, S, T) in enumerate(GEOM):
        lo = outs[l].reshape(Hp, Wp, 128)[1:H + 1, 1:W + 1, :80]
        logits.append(lo.transpose(2, 0, 1)[None])
        bc = outs[5 + l].reshape(Hp, Wp, 128)[1:H + 1, 1:W + 1, :5]
        bboxs.append(bc[..., 0:4].transpose(2, 0, 1)[None])
        ctrs.append(bc[..., 4:5].transpose(2, 0, 1)[None])
    return tuple(logits) + tuple(bboxs) + tuple(ctrs)


# speedup vs baseline: 1.4203x; 1.0432x over previous
import functools
import math

import jax
import jax.numpy as jnp
import numpy as np
from jax.experimental import pallas as pl
from jax.experimental.pallas import tpu as pltpu

C = 256
NCONV = 4
GROUPS = 32
EPS = 1e-5
LEVEL_HW = (64, 32, 16, 8, 4)

ACT_DT = jnp.bfloat16
MM_DT = jnp.bfloat16

GEOM = []
for _hw in LEVEL_HW:
    _Hp, _Wp = _hw + 2, _hw + 2
    _Np = _Hp * _Wp
    _S = _Wp + 1
    GEOM.append((_hw, _hw, _Hp, _Wp, _Np, _S, _Np + 2 * _S))
TMAX = max(g[6] for g in GEOM)


def _fcos_kernel(*refs):
    i = 0
    x_refs = refs[i:i + 5]; i += 5
    (cls_tw, cls_bb, cls_gg, cls_be,
     box_tw, box_bb, box_gg, box_be,
     hcls_w, hcls_b, hbc_w, hbc_b, gmat, svec, rsel) = refs[i:i + 15]
    i += 15
    m_refs = refs[i:i + 5]; i += 5
    lo_refs = refs[i:i + 5]; i += 5
    bo_refs = refs[i:i + 5]; i += 5
    X = refs[i]

    gmat_v = gmat[...]
    rsel_v = rsel[...]

    for l, (H, W, Hp, Wp, Np, S, T) in enumerate(GEOM):
        offs = [dy * Wp + dx for dy in (-1, 0, 1) for dx in (-1, 0, 1)]
        inv_cnt = 1.0 / float(H * W * (C // GROUPS))
        mask = m_refs[l][...]

        def conv9(wref, base, offs=offs, Np=Np, S=S):
            acc = None
            for t, off in enumerate(offs):
                xs = X[pl.ds(S + off, Np), :]
                p = jnp.dot(xs, wref[base + t],
                            preferred_element_type=jnp.float32)
                acc = p if acc is None else acc + p
            return acc

        def run_tower(twref, bref, gref, beref,
                      l=l, Np=Np, S=S, T=T, mask=mask, inv_cnt=inv_cnt,
                      conv9=conv9):
            X[pl.ds(0, T), :] = jnp.zeros((T, C), ACT_DT)
            X[pl.ds(S, Np), :] = x_refs[l][...]

            def body(i, carry):
                acc = conv9(twref, i * 9)
                y = (acc + bref[i]) * mask
                s1 = jnp.sum(y, axis=0, keepdims=True)
                s2 = jnp.sum(y * y, axis=0, keepdims=True)
                gs1 = jnp.dot(s1, gmat_v, preferred_element_type=jnp.float32)
                gs2 = jnp.dot(s2, gmat_v, preferred_element_type=jnp.float32)
                mean = gs1 * inv_cnt
                var = gs2 * inv_cnt - mean * mean
                rstd = jax.lax.rsqrt(var + EPS)
                z = (y - mean) * (rstd * gref[i]) + beref[i]
                z = jnp.maximum(z, 0.0) * mask
                X[pl.ds(S, Np), :] = z.astype(ACT_DT)
                return carry

            jax.lax.fori_loop(0, NCONV, body, 0, unroll=True)

        run_tower(cls_tw, cls_bb, cls_gg, cls_be)
        lo_refs[l][...] = conv9(hcls_w, 0) + hcls_b[...]

        run_tower(box_tw, box_bb, box_gg, box_be)
        v = (conv9(hbc_w, 0) + hbc_b[...]) * svec[l]
        bo_refs[l][...] = rsel_v * jnp.maximum(v, 0.0) + (1.0 - rsel_v) * v


def _tower_taps(w):
    return w.transpose(0, 3, 4, 2, 1).reshape(NCONV * 9, C, C).astype(MM_DT)


def _head_taps(w, pad_to=128):
    o = w.shape[0]
    t = w.transpose(2, 3, 1, 0).reshape(9, C, o)
    return jnp.pad(t, ((0, 0), (0, 0), (0, pad_to - o))).astype(MM_DT)


@functools.partial(jax.jit)
def kernel(p3, p4, p5, p6, p7,
           cls_w, cls_b, cls_gn_g, cls_gn_b,
           bbox_w, bbox_b, bbox_gn_g, bbox_gn_b,
           head_cls_w, head_cls_b, head_bbox_w, head_bbox_b,
           head_ctr_w, head_ctr_b, scales):
    feats = (p3, p4, p5, p6, p7)
    xs, masks = [], []
    for (H, W, Hp, Wp, Np, S, T), f in zip(GEOM, feats):
        x = jnp.pad(f[0].transpose(1, 2, 0), ((1, 1), (1, 1), (0, 0)))
        xs.append(x.reshape(Np, C).astype(ACT_DT))
        m = np.zeros((Hp, Wp, 1), np.float32)
        m[1:H + 1, 1:W + 1] = 1.0
        masks.append(jnp.asarray(m.reshape(Np, 1)))

    cls_tw = _tower_taps(cls_w)
    box_tw = _tower_taps(bbox_w)
    per_layer = lambda a: a.reshape(NCONV, 1, C).astype(jnp.float32)
    cls_bb, cls_gg, cls_be = per_layer(cls_b), per_layer(cls_gn_g), per_layer(cls_gn_b)
    box_bb, box_gg, box_be = per_layer(bbox_b), per_layer(bbox_gn_g), per_layer(bbox_gn_b)

    hcls_w = _head_taps(head_cls_w)
    hcls_b = jnp.pad(head_cls_b, (0, 128 - 80)).reshape(1, 128).astype(jnp.float32)
    hbc_w = _head_taps(jnp.concatenate([head_bbox_w, head_ctr_w], axis=0))
    hbc_b = jnp.pad(jnp.concatenate([head_bbox_b, head_ctr_b]), (0, 128 - 5)
                    ).reshape(1, 128).astype(jnp.float32)

    gmat = jnp.asarray(np.kron(np.eye(GROUPS, dtype=np.float32),
                               np.ones((C // GROUPS, C // GROUPS), np.float32)))
    lane = np.arange(128)
    svec = jnp.where(jnp.asarray(lane[None, None, :] < 4),
                     scales[:, None, None].astype(jnp.float32), 1.0)
    rsel = jnp.asarray((lane[None, :] < 4).astype(np.float32))

    out_shape = ([jax.ShapeDtypeStruct((g[4], 128), jnp.float32) for g in GEOM]
                 + [jax.ShapeDtypeStruct((g[4], 128), jnp.float32) for g in GEOM])

    outs = pl.pallas_call(
        _fcos_kernel,
        out_shape=out_shape,
        scratch_shapes=[pltpu.VMEM((TMAX, C), ACT_DT)],
    )(*xs, cls_tw, cls_bb, cls_gg, cls_be, box_tw, box_bb, box_gg, box_be,
      hcls_w, hcls_b, hbc_w, hbc_b, gmat, svec, rsel, *masks)

    logits, bboxs, ctrs = [], [], []
    for l, (H, W, Hp, Wp, Np, S, T) in enumerate(GEOM):
        lo = outs[l].reshape(Hp, Wp, 128)[1:H + 1, 1:W + 1, :80]
        logits.append(lo.transpose(2, 0, 1)[None])
        bc = outs[5 + l].reshape(Hp, Wp, 128)[1:H + 1, 1:W + 1, :5]
        bboxs.append(bc[..., 0:4].transpose(2, 0, 1)[None])
        ctrs.append(bc[..., 4:5].transpose(2, 0, 1)[None])
    return tuple(logits) + tuple(bboxs) + tuple(ctrs)


# speedup vs baseline: 1.8431x; 1.2977x over previous
import functools
import math

import jax
import jax.numpy as jnp
import numpy as np
from jax.experimental import pallas as pl
from jax.experimental.pallas import tpu as pltpu

C = 256
NCONV = 4
GROUPS = 32
EPS = 1e-5
LEVEL_HW = (64, 32, 16, 8, 4)

ACT_DT = jnp.bfloat16
MM_DT = jnp.bfloat16

GEOM = []
for _hw in LEVEL_HW:
    _Hp, _Wp = _hw + 2, _hw + 2
    _Np = _Hp * _Wp
    _S = _Wp + 1
    GEOM.append((_hw, _hw, _Hp, _Wp, _Np, _S, _Np + 2 * _S))
TMAX = max(g[6] for g in GEOM)


def _fcos_kernel(*refs):
    i = 0
    x_refs = refs[i:i + 5]; i += 5
    (cls_tw, cls_bb, cls_gg, cls_be,
     box_tw, box_bb, box_gg, box_be,
     hcls_w, hcls_b, hbc_w, hbc_b, gmat, svec, rsel) = refs[i:i + 15]
    i += 15
    m_refs = refs[i:i + 5]; i += 5
    lo_refs = refs[i:i + 5]; i += 5
    bo_refs = refs[i:i + 5]; i += 5
    XA, XB = refs[i], refs[i + 1]

    gmat_v = gmat[...]
    rsel_v = rsel[...]

    for l, (H, W, Hp, Wp, Np, S, T) in enumerate(GEOM):
        offs = [dy * Wp + dx for dy in (-1, 0, 1) for dx in (-1, 0, 1)]
        inv_cnt = 1.0 / float(H * W * (C // GROUPS))
        mask = m_refs[l][...]

        def conv9(X, wref, base, offs=offs, Np=Np, S=S):
            acc = None
            for t, off in enumerate(offs):
                xs = X[pl.ds(S + off, Np), :]
                p = jnp.dot(xs, wref[base + t],
                            preferred_element_type=jnp.float32)
                acc = p if acc is None else acc + p
            return acc

        def gn_relu(acc, bref, gref, beref, i,
                    mask=mask, inv_cnt=inv_cnt):
            ym = (acc + bref[i]) * mask
            s1 = jnp.sum(ym, axis=0, keepdims=True)
            s2 = jnp.sum(ym * ym, axis=0, keepdims=True)
            gs1 = jnp.dot(s1, gmat_v, preferred_element_type=jnp.float32)
            gs2 = jnp.dot(s2, gmat_v, preferred_element_type=jnp.float32)
            mean = gs1 * inv_cnt
            var = gs2 * inv_cnt - mean * mean
            a = jax.lax.rsqrt(var + EPS) * gref[i]
            c = beref[i] - mean * a
            z = jnp.maximum(ym * a + c, 0.0) * mask
            return z.astype(ACT_DT)

        x0 = x_refs[l][...]
        for X in (XA, XB):
            X[pl.ds(0, T), :] = jnp.zeros((T, C), ACT_DT)
            X[pl.ds(S, Np), :] = x0
        for i in range(NCONV):
            accA = conv9(XA, cls_tw, i * 9)
            accB = conv9(XB, box_tw, i * 9)
            XA[pl.ds(S, Np), :] = gn_relu(accA, cls_bb, cls_gg, cls_be, i)
            XB[pl.ds(S, Np), :] = gn_relu(accB, box_bb, box_gg, box_be, i)

        lo_refs[l][...] = conv9(XA, hcls_w, 0) + hcls_b[...]
        v = (conv9(XB, hbc_w, 0) + hbc_b[...]) * svec[l]
        bo_refs[l][...] = rsel_v * jnp.maximum(v, 0.0) + (1.0 - rsel_v) * v


def _tower_taps(w):
    return w.transpose(0, 3, 4, 2, 1).reshape(NCONV * 9, C, C).astype(MM_DT)


def _head_taps(w, pad_to=128):
    o = w.shape[0]
    t = w.transpose(2, 3, 1, 0).reshape(9, C, o)
    return jnp.pad(t, ((0, 0), (0, 0), (0, pad_to - o))).astype(MM_DT)


@functools.partial(jax.jit)
def kernel(p3, p4, p5, p6, p7,
           cls_w, cls_b, cls_gn_g, cls_gn_b,
           bbox_w, bbox_b, bbox_gn_g, bbox_gn_b,
           head_cls_w, head_cls_b, head_bbox_w, head_bbox_b,
           head_ctr_w, head_ctr_b, scales):
    feats = (p3, p4, p5, p6, p7)
    xs, masks = [], []
    for (H, W, Hp, Wp, Np, S, T), f in zip(GEOM, feats):
        x = jnp.pad(f[0].transpose(1, 2, 0), ((1, 1), (1, 1), (0, 0)))
        xs.append(x.reshape(Np, C).astype(ACT_DT))
        m = np.zeros((Hp, Wp, 1), np.float32)
        m[1:H + 1, 1:W + 1] = 1.0
        masks.append(jnp.asarray(m.reshape(Np, 1)))

    cls_tw = _tower_taps(cls_w)
    box_tw = _tower_taps(bbox_w)
    per_layer = lambda a: a.reshape(NCONV, 1, C).astype(jnp.float32)
    cls_bb, cls_gg, cls_be = per_layer(cls_b), per_layer(cls_gn_g), per_layer(cls_gn_b)
    box_bb, box_gg, box_be = per_layer(bbox_b), per_layer(bbox_gn_g), per_layer(bbox_gn_b)

    hcls_w = _head_taps(head_cls_w)
    hcls_b = jnp.pad(head_cls_b, (0, 128 - 80)).reshape(1, 128).astype(jnp.float32)
    hbc_w = _head_taps(jnp.concatenate([head_bbox_w, head_ctr_w], axis=0))
    hbc_b = jnp.pad(jnp.concatenate([head_bbox_b, head_ctr_b]), (0, 128 - 5)
                    ).reshape(1, 128).astype(jnp.float32)

    gmat = jnp.asarray(np.kron(np.eye(GROUPS, dtype=np.float32),
                               np.ones((C // GROUPS, C // GROUPS), np.float32)))
    lane = np.arange(128)
    svec = jnp.where(jnp.asarray(lane[None, None, :] < 4),
                     scales[:, None, None].astype(jnp.float32), 1.0)
    rsel = jnp.asarray((lane[None, :] < 4).astype(np.float32))

    out_shape = ([jax.ShapeDtypeStruct((g[4], 128), jnp.float32) for g in GEOM]
                 + [jax.ShapeDtypeStruct((g[4], 128), jnp.float32) for g in GEOM])

    outs = pl.pallas_call(
        _fcos_kernel,
        out_shape=out_shape,
        scratch_shapes=[pltpu.VMEM((TMAX, C), ACT_DT),
                        pltpu.VMEM((TMAX, C), ACT_DT)],
    )(*xs, cls_tw, cls_bb, cls_gg, cls_be, box_tw, box_bb, box_gg, box_be,
      hcls_w, hcls_b, hbc_w, hbc_b, gmat, svec, rsel, *masks)

    logits, bboxs, ctrs = [], [], []
    for l, (H, W, Hp, Wp, Np, S, T) in enumerate(GEOM):
        lo = outs[l].reshape(Hp, Wp, 128)[1:H + 1, 1:W + 1, :80]
        logits.append(lo.transpose(2, 0, 1)[None])
        bc = outs[5 + l].reshape(Hp, Wp, 128)[1:H + 1, 1:W + 1, :5]
        bboxs.append(bc[..., 0:4].transpose(2, 0, 1)[None])
        ctrs.append(bc[..., 4:5].transpose(2, 0, 1)[None])
    return tuple(logits) + tuple(bboxs) + tuple(ctrs)


# speedup vs baseline: 2.0176x; 1.0947x over previous
import functools
import math

import jax
import jax.numpy as jnp
import numpy as np
from jax.experimental import pallas as pl
from jax.experimental.pallas import tpu as pltpu

C = 256
NCONV = 4
GROUPS = 32
EPS = 1e-5
LEVEL_HW = (64, 32, 16, 8, 4)

ACT_DT = jnp.bfloat16
MM_DT = jnp.bfloat16

GEOM = []
for _hw in LEVEL_HW:
    _Hp, _Wp = _hw + 2, _hw + 2
    _Np = _Hp * _Wp
    _S = _Wp + 1
    GEOM.append((_hw, _hw, _Hp, _Wp, _Np, _S, _Np + 2 * _S))
TMAX = max(g[6] for g in GEOM)


def _fcos_kernel(*refs):
    i = 0
    x_refs = refs[i:i + 5]; i += 5
    (cls_tw, cls_bb, cls_gg, cls_be,
     box_tw, box_bb, box_gg, box_be,
     hcls_w, hcls_b, hbc_w, hbc_b, gmat, svec, rsel) = refs[i:i + 15]
    i += 15
    m_refs = refs[i:i + 5]; i += 5
    lo_refs = refs[i:i + 5]; i += 5
    bo_refs = refs[i:i + 5]; i += 5
    scr = refs[i:i + 10]

    gmat_v = gmat[...]
    rsel_v = rsel[...]

    units = []
    for l, (H, W, Hp, Wp, Np, S, T) in enumerate(GEOM):
        offs = [dy * Wp + dx for dy in (-1, 0, 1) for dx in (-1, 0, 1)]
        inv_cnt = 1.0 / float(H * W * (C // GROUPS))
        for t_idx, (tw, bb, gg, be) in enumerate(
                ((cls_tw, cls_bb, cls_gg, cls_be),
                 (box_tw, box_bb, box_gg, box_be))):
            units.append(dict(l=l, X=scr[2 * l + t_idx], offs=offs, Np=Np,
                              S=S, T=T, inv_cnt=inv_cnt, tw=tw, bb=bb,
                              gg=gg, be=be))

    def conv9(u, wref, base):
        X, S, Np = u['X'], u['S'], u['Np']
        acc = None
        for t, off in enumerate(u['offs']):
            xs = X[pl.ds(S + off, Np), :]
            p = jnp.dot(xs, wref[base + t],
                        preferred_element_type=jnp.float32)
            acc = p if acc is None else acc + p
        return acc

    def gn_relu(u, acc, i):
        mask = m_refs[u['l']][...]
        ym = (acc + u['bb'][i]) * mask
        s1 = jnp.sum(ym, axis=0, keepdims=True)
        s2 = jnp.sum(ym * ym, axis=0, keepdims=True)
        gs1 = jnp.dot(s1, gmat_v, preferred_element_type=jnp.float32)
        gs2 = jnp.dot(s2, gmat_v, preferred_element_type=jnp.float32)
        mean = gs1 * u['inv_cnt']
        var = gs2 * u['inv_cnt'] - mean * mean
        a = jax.lax.rsqrt(var + EPS) * u['gg'][i]
        c = u['be'][i] - mean * a
        return (jnp.maximum(ym * a + c, 0.0) * mask).astype(ACT_DT)

    for u in units:
        u['X'][pl.ds(0, u['T']), :] = jnp.zeros((u['T'], C), ACT_DT)
        u['X'][pl.ds(u['S'], u['Np']), :] = x_refs[u['l']][...]

    pend = None
    for i in range(NCONV):
        for u in units:
            acc = conv9(u, u['tw'], i * 9)
            if pend is not None:
                pu, pacc, pi = pend
                pu['X'][pl.ds(pu['S'], pu['Np']), :] = gn_relu(pu, pacc, pi)
            pend = (u, acc, i)
    pu, pacc, pi = pend
    pu['X'][pl.ds(pu['S'], pu['Np']), :] = gn_relu(pu, pacc, pi)

    for l in range(5):
        uA, uB = units[2 * l], units[2 * l + 1]
        lo_refs[l][...] = (conv9(uA, hcls_w, 0) + hcls_b[...]).astype(ACT_DT)
        v = (conv9(uB, hbc_w, 0) + hbc_b[...]) * svec[l]
        bo_refs[l][...] = (rsel_v * jnp.maximum(v, 0.0)
                           + (1.0 - rsel_v) * v).astype(ACT_DT)


def _tower_taps(w):
    return w.transpose(0, 3, 4, 2, 1).reshape(NCONV * 9, C, C).astype(MM_DT)


def _head_taps(w, pad_to=128):
    o = w.shape[0]
    t = w.transpose(2, 3, 1, 0).reshape(9, C, o)
    return jnp.pad(t, ((0, 0), (0, 0), (0, pad_to - o))).astype(MM_DT)


@functools.partial(jax.jit)
def kernel(p3, p4, p5, p6, p7,
           cls_w, cls_b, cls_gn_g, cls_gn_b,
           bbox_w, bbox_b, bbox_gn_g, bbox_gn_b,
           head_cls_w, head_cls_b, head_bbox_w, head_bbox_b,
           head_ctr_w, head_ctr_b, scales):
    feats = (p3, p4, p5, p6, p7)
    xs, masks = [], []
    for (H, W, Hp, Wp, Np, S, T), f in zip(GEOM, feats):
        x = jnp.pad(f[0].transpose(1, 2, 0), ((1, 1), (1, 1), (0, 0)))
        xs.append(x.reshape(Np, C).astype(ACT_DT))
        m = np.zeros((Hp, Wp, 1), np.float32)
        m[1:H + 1, 1:W + 1] = 1.0
        masks.append(jnp.asarray(m.reshape(Np, 1)))

    cls_tw = _tower_taps(cls_w)
    box_tw = _tower_taps(bbox_w)
    per_layer = lambda a: a.reshape(NCONV, 1, C).astype(jnp.float32)
    cls_bb, cls_gg, cls_be = per_layer(cls_b), per_layer(cls_gn_g), per_layer(cls_gn_b)
    box_bb, box_gg, box_be = per_layer(bbox_b), per_layer(bbox_gn_g), per_layer(bbox_gn_b)

    hcls_w = _head_taps(head_cls_w)
    hcls_b = jnp.pad(head_cls_b, (0, 128 - 80)).reshape(1, 128).astype(jnp.float32)
    hbc_w = _head_taps(jnp.concatenate([head_bbox_w, head_ctr_w], axis=0))
    hbc_b = jnp.pad(jnp.concatenate([head_bbox_b, head_ctr_b]), (0, 128 - 5)
                    ).reshape(1, 128).astype(jnp.float32)

    gmat = jnp.asarray(np.kron(np.eye(GROUPS, dtype=np.float32),
                               np.ones((C // GROUPS, C // GROUPS), np.float32)))
    lane = np.arange(128)
    svec = jnp.where(jnp.asarray(lane[None, None, :] < 4),
                     scales[:, None, None].astype(jnp.float32), 1.0)
    rsel = jnp.asarray((lane[None, :] < 4).astype(np.float32))

    out_shape = ([jax.ShapeDtypeStruct((g[4], 128), ACT_DT) for g in GEOM]
                 + [jax.ShapeDtypeStruct((g[4], 128), ACT_DT) for g in GEOM])

    outs = pl.pallas_call(
        _fcos_kernel,
        out_shape=out_shape,
        scratch_shapes=[pltpu.VMEM((g[6], C), ACT_DT)
                        for g in GEOM for _ in (0, 1)],
    )(*xs, cls_tw, cls_bb, cls_gg, cls_be, box_tw, box_bb, box_gg, box_be,
      hcls_w, hcls_b, hbc_w, hbc_b, gmat, svec, rsel, *masks)

    logits, bboxs, ctrs = [], [], []
    for l, (H, W, Hp, Wp, Np, S, T) in enumerate(GEOM):
        lo = outs[l].astype(jnp.float32).reshape(Hp, Wp, 128)[1:H + 1, 1:W + 1, :80]
        logits.append(lo.transpose(2, 0, 1)[None])
        bc = outs[5 + l].astype(jnp.float32).reshape(Hp, Wp, 128)[1:H + 1, 1:W + 1, :5]
        bboxs.append(bc[..., 0:4].transpose(2, 0, 1)[None])
        ctrs.append(bc[..., 4:5].transpose(2, 0, 1)[None])
    return tuple(logits) + tuple(bboxs) + tuple(ctrs)
